# baseline (speedup 1.0000x reference)
import jax
import jax.numpy as jnp
from jax.experimental import pallas as pl
from jax.experimental.pallas import tpu as pltpu

_B = 16
_NODES = 512
_HID = 128
_EPS = 1e-5


def _bf_dot(x, w):
    return jnp.dot(x.astype(jnp.bfloat16), w.astype(jnp.bfloat16),
                   preferred_element_type=jnp.float32)


def _bn_cols(x, w, b):
    m = jnp.mean(x, axis=0, keepdims=True)
    v = jnp.mean((x - m) ** 2, axis=0, keepdims=True)
    return (x - m) / jnp.sqrt(v + _EPS) * w + b


def _leaky(x):
    return jnp.where(x > 0, x, 0.1 * x)


def _fwd(adj_ref, sn_ref,
         bin_w, bin_b,
         w1_0, b1_0, w2_0, b2_0, bnw_0, bnb_0,
         w1_1, b1_1, w2_1, b2_1, bnw_1, bnb_1,
         w1_2, b1_2, w2_2, b2_2, bnw_2, bnb_2,
         ln_w, ln_b,
         fw1, fb1, n1w, n1b, fw2, fb2, n2w, n2b, fw3, fb3,
         out_ref, a_scr, z_scr):
    f32 = jnp.float32
    b = pl.program_id(0)

    ab4 = adj_ref[...].astype(jnp.bfloat16)
    a_scr[pl.ds(b * 4, 4)] = ab4

    h0 = _bn_cols(sn_ref[...], bin_w[...], bin_b[...])
    h0b = h0.astype(jnp.bfloat16)
    ag4 = jnp.stack([jax.lax.dot_general(
        ab4[j], h0b, (((0,), (0,)), ((), ())),
        preferred_element_type=f32) for j in range(4)], axis=0)
    z4 = (h0[None] + ag4).reshape(4 * _NODES, 3)
    z4 = _bf_dot(z4, w1_0[...]) + b1_0[...]
    z4 = jax.nn.relu(z4)
    z4 = _bf_dot(z4, w2_0[...]) + b2_0[...]
    z_scr[pl.ds(b * 4, 4)] = z4.reshape(4, _NODES, _HID)

    @pl.when(b == _B // 4 - 1)
    def _compute():
        def aggr(h):
            outs = []
            for i in range(_B):
                outs.append(jax.lax.dot_general(
                    a_scr[i], h[i].astype(jnp.bfloat16),
                    (((0,), (0,)), ((), ())),
                    preferred_element_type=f32))
            return jnp.stack(outs, axis=0)

        z = z_scr[...].reshape(_B * _NODES, _HID)
        z = _bn_cols(z, bnw_0[...], bnb_0[...])
        h = jax.nn.relu(z).reshape(_B, _NODES, _HID)

        for w1, b1, w2, b2, bnw, bnb in (
                (w1_1, b1_1, w2_1, b2_1, bnw_1, bnb_1),
                (w1_2, b1_2, w2_2, b2_2, bnw_2, bnb_2)):
            w1b, w2b = w1[...].astype(jnp.bfloat16), w2[...].astype(jnp.bfloat16)
            chunks = []
            for c in range(0, _B, 4):
                t = jnp.stack(
                    [h[c + j] + jax.lax.dot_general(
                        a_scr[c + j], h[c + j].astype(jnp.bfloat16),
                        (((0,), (0,)), ((), ())),
                        preferred_element_type=f32) for j in range(4)],
                    axis=0).reshape(4 * _NODES, _HID)
                zc = jnp.dot(t.astype(jnp.bfloat16), w1b,
                             preferred_element_type=f32) + b1[...]
                zc = _leaky(zc)
                zc = jnp.dot(zc.astype(jnp.bfloat16), w2b,
                             preferred_element_type=f32) + b2[...]
                chunks.append(zc)
            zz = jnp.concatenate(chunks, axis=0)
            zz = _bn_cols(zz, bnw[...], bnb[...])
            h = jax.nn.relu(zz + h.reshape(_B * _NODES, _HID)).reshape(
                _B, _NODES, _HID)

        pooled = jnp.mean(h, axis=1)
        pm = jnp.mean(pooled, axis=-1, keepdims=True)
        pv = jnp.mean((pooled - pm) ** 2, axis=-1, keepdims=True)
        emb = (pooled - pm) / jnp.sqrt(pv + _EPS) * ln_w[...] + ln_b[...]

        y = _bf_dot(emb, fw1[...]) + fb1[...]
        y = _leaky(_bn_cols(y, n1w[...], n1b[...]))
        y = _bf_dot(y, fw2[...]) + fb2[...]
        y = _leaky(_bn_cols(y, n2w[...], n2b[...]))
        yb = y.astype(jnp.bfloat16).astype(f32)
        wb = fw3[...].astype(jnp.bfloat16).astype(f32)
        out_ref[...] = (jnp.sum(yb * wb, axis=-1, keepdims=True)
                        + fb3[...])


def kernel(adjacency_matrices, single_nodes, params):
    p = params
    row = lambda a: a.reshape(1, -1)
    flat = [p['bn_in_w'].reshape(1, 3), p['bn_in_b'].reshape(1, 3)]
    for lp in p['layers']:
        flat += [lp['W1'], row(lp['b1']), lp['W2'], row(lp['b2']),
                 row(lp['bn_w']), row(lp['bn_b'])]
    fc = p['fc']
    flat += [row(p['ln_w']), row(p['ln_b']),
             fc['W1'], row(fc['b1']), row(fc['n1w']), row(fc['n1b']),
             fc['W2'], row(fc['b2']), row(fc['n2w']), row(fc['n2b']),
             fc['W3'].reshape(1, -1), row(fc['b3'])]

    full = lambda a: pl.BlockSpec(a.shape, lambda b: (0,) * a.ndim)
    return pl.pallas_call(
        _fwd,
        grid=(_B // 4,),
        in_specs=[pl.BlockSpec((4, _NODES, _NODES), lambda b: (b, 0, 0)),
                  full(single_nodes)] + [full(a) for a in flat],
        out_specs=pl.BlockSpec((_B, 1), lambda b: (0, 0)),
        out_shape=jax.ShapeDtypeStruct((_B, 1), jnp.float32),
        scratch_shapes=[pltpu.VMEM((_B, _NODES, _NODES), jnp.bfloat16),
                        pltpu.VMEM((_B, _NODES, _HID), jnp.float32)],
        compiler_params=pltpu.CompilerParams(
            vmem_limit_bytes=60 * 1024 * 1024),
    )(adjacency_matrices, single_nodes, *flat)


# speedup vs baseline: 1.0325x; 1.0325x over previous
import jax
import jax.numpy as jnp
from jax.experimental import pallas as pl
from jax.experimental.pallas import tpu as pltpu

_B = 16
_NODES = 512
_HID = 128
_EPS = 1e-5


def _bf_dot(x, w):
    return jnp.dot(x.astype(jnp.bfloat16), w.astype(jnp.bfloat16),
                   preferred_element_type=jnp.float32)


def _bn_cols(x, w, b):
    m = jnp.mean(x, axis=0, keepdims=True)
    v = jnp.mean((x - m) ** 2, axis=0, keepdims=True)
    return (x - m) / jnp.sqrt(v + _EPS) * w + b


def _leaky(x):
    return jnp.where(x > 0, x, 0.1 * x)


def _fwd(adj_ref, sn_ref,
         bin_w, bin_b,
         w1_0, b1_0, w2_0, b2_0, bnw_0, bnb_0,
         w1_1, b1_1, w2_1, b2_1, bnw_1, bnb_1,
         w1_2, b1_2, w2_2, b2_2, bnw_2, bnb_2,
         ln_w, ln_b,
         fw1, fb1, n1w, n1b, fw2, fb2, n2w, n2b, fw3, fb3,
         out_ref, a_scr, ag_scr):
    f32 = jnp.float32
    b = pl.program_id(0)

    ab4 = adj_ref[...].astype(jnp.bfloat16)
    a_scr[pl.ds(b * 4, 4)] = ab4

    h0 = _bn_cols(sn_ref[...], bin_w[...], bin_b[...])
    h0b = h0.astype(jnp.bfloat16)
    for j in range(4):
        ag_scr[pl.ds(b * 4 + j, 1)] = jax.lax.dot_general(
            ab4[j], h0b, (((0,), (0,)), ((), ())),
            preferred_element_type=f32)[None]

    @pl.when(b == _B // 4 - 1)
    def _compute():
        def aggr(h):
            outs = []
            for i in range(_B):
                outs.append(jax.lax.dot_general(
                    a_scr[i], h[i].astype(jnp.bfloat16),
                    (((0,), (0,)), ((), ())),
                    preferred_element_type=f32))
            return jnp.stack(outs, axis=0)

        z = (h0[None] + ag_scr[...]).reshape(_B * _NODES, 3)
        z = _bf_dot(z, w1_0[...]) + b1_0[...]
        z = jax.nn.relu(z)
        z = _bf_dot(z, w2_0[...]) + b2_0[...]
        z = _bn_cols(z, bnw_0[...], bnb_0[...])
        h = jax.nn.relu(z).reshape(_B, _NODES, _HID)

        for w1, b1, w2, b2, bnw, bnb in (
                (w1_1, b1_1, w2_1, b2_1, bnw_1, bnb_1),
                (w1_2, b1_2, w2_2, b2_2, bnw_2, bnb_2)):
            w1b, w2b = w1[...].astype(jnp.bfloat16), w2[...].astype(jnp.bfloat16)
            chunks = []
            for c in range(0, _B, 4):
                t = jnp.stack(
                    [h[c + j] + jax.lax.dot_general(
                        a_scr[c + j], h[c + j].astype(jnp.bfloat16),
                        (((0,), (0,)), ((), ())),
                        preferred_element_type=f32) for j in range(4)],
                    axis=0).reshape(4 * _NODES, _HID)
                zc = jnp.dot(t.astype(jnp.bfloat16), w1b,
                             preferred_element_type=f32) + b1[...]
                zc = _leaky(zc)
                zc = jnp.dot(zc.astype(jnp.bfloat16), w2b,
                             preferred_element_type=f32) + b2[...]
                chunks.append(zc)
            zz = jnp.concatenate(chunks, axis=0)
            zz = _bn_cols(zz, bnw[...], bnb[...])
            h = jax.nn.relu(zz + h.reshape(_B * _NODES, _HID)).reshape(
                _B, _NODES, _HID)

        pooled = jnp.mean(h, axis=1)
        pm = jnp.mean(pooled, axis=-1, keepdims=True)
        pv = jnp.mean((pooled - pm) ** 2, axis=-1, keepdims=True)
        emb = (pooled - pm) / jnp.sqrt(pv + _EPS) * ln_w[...] + ln_b[...]

        y = _bf_dot(emb, fw1[...]) + fb1[...]
        y = _leaky(_bn_cols(y, n1w[...], n1b[...]))
        y = _bf_dot(y, fw2[...]) + fb2[...]
        y = _leaky(_bn_cols(y, n2w[...], n2b[...]))
        yb = y.astype(jnp.bfloat16).astype(f32)
        wb = fw3[...].astype(jnp.bfloat16).astype(f32)
        out_ref[...] = (jnp.sum(yb * wb, axis=-1, keepdims=True)
                        + fb3[...])


def kernel(adjacency_matrices, single_nodes, params):
    p = params
    row = lambda a: a.reshape(1, -1)
    flat = [p['bn_in_w'].reshape(1, 3), p['bn_in_b'].reshape(1, 3)]
    for lp in p['layers']:
        flat += [lp['W1'], row(lp['b1']), lp['W2'], row(lp['b2']),
                 row(lp['bn_w']), row(lp['bn_b'])]
    fc = p['fc']
    flat += [row(p['ln_w']), row(p['ln_b']),
             fc['W1'], row(fc['b1']), row(fc['n1w']), row(fc['n1b']),
             fc['W2'], row(fc['b2']), row(fc['n2w']), row(fc['n2b']),
             fc['W3'].reshape(1, -1), row(fc['b3'])]

    full = lambda a: pl.BlockSpec(a.shape, lambda b: (0,) * a.ndim)
    return pl.pallas_call(
        _fwd,
        grid=(_B // 4,),
        in_specs=[pl.BlockSpec((4, _NODES, _NODES), lambda b: (b, 0, 0)),
                  full(single_nodes)] + [full(a) for a in flat],
        out_specs=pl.BlockSpec((_B, 1), lambda b: (0, 0)),
        out_shape=jax.ShapeDtypeStruct((_B, 1), jnp.float32),
        scratch_shapes=[pltpu.VMEM((_B, _NODES, _NODES), jnp.bfloat16),
                        pltpu.VMEM((_B, _NODES, 3), jnp.float32)],
        compiler_params=pltpu.CompilerParams(
            vmem_limit_bytes=60 * 1024 * 1024),
    )(adjacency_matrices, single_nodes, *flat)
